# baseline (speedup 1.0000x reference)
import functools

import jax
import jax.numpy as jnp
from jax import lax
from jax.experimental import pallas as pl
from jax.experimental.pallas import tpu as pltpu
from jax.experimental.pallas import tpu_sc as plsc

NC = 2
NS = 16
NW = NC * NS


def _make_segsum(N, E, D):
  ch = 80
  nbuf = 3
  ep = -(-(E // NW) // (ch * nbuf)) * ch * nbuf
  npass = ep // (ch * nbuf)
  na = -(-N // ch) * ch + ch
  rch = 80
  nrc = N // rch
  nrc_per_tile = -(-nrc // NS)
  nzc = na // ch
  nzc_per_tile = -(-nzc // NS)

  mesh = plsc.VectorSubcoreMesh(core_axis_name="c", subcore_axis_name="s")

  @functools.partial(
      pl.kernel,
      out_type=jax.ShapeDtypeStruct((NC, N, D), jnp.float32),
      mesh=mesh,
      scratch_types=[
          [pltpu.VMEM((nbuf, ch), jnp.int32)] * 2,
          [pltpu.VMEM((nbuf, ch), jnp.int32)] * 2,
          [pltpu.VMEM((ch, D), jnp.float32)] * nbuf,
          pltpu.VMEM_SHARED((na, D), jnp.float32),
          [pltpu.SemaphoreType.DMA] * 2,
          [pltpu.SemaphoreType.DMA] * nbuf,
          [pltpu.SemaphoreType.DMA] * nbuf,
      ],
  )
  def segsum(src_hbm, dst_hbm, x_hbm, out_hbm, sidxb, didxb, rows, yacc,
             isem, gsem, ssem):
    c = lax.axis_index("c")
    s = lax.axis_index("s")
    wid = c * NS + s

    pltpu.async_copy(src_hbm.at[wid, 0], sidxb[0], isem[0])
    pltpu.async_copy(dst_hbm.at[wid, 0], didxb[0], isem[0])

    zv = jnp.zeros((16,), jnp.float32)

    def zrow(r, carry):
      for k in range(D // 16):
        rows[0][r, pl.ds(k * 16, 16)] = zv
      return carry

    lax.fori_loop(0, ch, zrow, 0)

    for z in range(nzc_per_tile):
      ci = s + NS * z

      @pl.when(ci < nzc)
      def _():
        pltpu.sync_copy(rows[0],
                        yacc.at[pl.ds(pl.multiple_of(ci * ch, 8), ch)])

    plsc.subcore_barrier()

    def two_passes(u, carry):
      for q in range(2):
        t = 2 * u + q
        nxt = 1 - q

        @pl.when(t + 1 < npass)
        def _():
          pltpu.async_copy(src_hbm.at[wid, t + 1], sidxb[nxt], isem[nxt])
          pltpu.async_copy(dst_hbm.at[wid, t + 1], didxb[nxt], isem[nxt])

        pltpu.make_async_copy(src_hbm.at[wid, t], sidxb[q], isem[q]).wait()
        pltpu.make_async_copy(dst_hbm.at[wid, t], didxb[q], isem[q]).wait()

        gh = [pltpu.async_copy(x_hbm.at[sidxb[q].at[b]], rows[b], gsem[b])
              for b in range(nbuf)]
        sh = []
        for b in range(nbuf):
          gh[b].wait()
          sh.append(pltpu.async_copy(rows[b], yacc.at[didxb[q].at[b]],
                                     ssem[b], add=True))
        for b in range(nbuf):
          sh[b].wait()
      return carry

    lax.fori_loop(0, npass // 2, two_passes, 0)

    plsc.subcore_barrier()

    for z in range(nrc_per_tile):
      ci = s + NS * z

      @pl.when(ci < nrc)
      def _():
        r0 = pl.multiple_of(ci * rch, 8)
        pltpu.sync_copy(yacc.at[pl.ds(r0, rch)], out_hbm.at[c, pl.ds(r0, rch)])

  return segsum


def _dense_body(yp_ref, x_ref, w1_ref, b1_ref, w2_ref, b2_ref, eps_ref,
                gamma_ref, beta_ref, o_ref):
  n = x_ref.shape[0]
  y = yp_ref[0] + yp_ref[1]
  h = y + (1.0 + eps_ref[0]) * x_ref[...]
  h = lax.dot_general(h, w1_ref[...], (((1,), (1,)), ((), ())),
                      preferred_element_type=jnp.float32)
  h = jnp.maximum(h + b1_ref[...][None, :], 0.0)
  h = lax.dot_general(h, w2_ref[...], (((1,), (1,)), ((), ())),
                      preferred_element_type=jnp.float32)
  h = h + b2_ref[...][None, :]
  mean = jnp.sum(h, axis=0, keepdims=True) * (1.0 / n)
  d = h - mean
  var = jnp.sum(d * d, axis=0, keepdims=True) * (1.0 / n)
  o_ref[...] = d * lax.rsqrt(var + 1e-5) * gamma_ref[...][None, :] \
      + beta_ref[...][None, :]


def kernel(x, edge_index, W1, b1, W2, b2, eps, gamma, beta):
  N, D = x.shape
  E = edge_index.shape[1]
  ch, nbuf = 80, 3
  ep = -(-(E // NW) // (ch * nbuf)) * ch * nbuf
  pad = ep * NW - E
  src = jnp.concatenate([edge_index[0], jnp.zeros((pad,), jnp.int32)])
  dst = jnp.concatenate([edge_index[1], jnp.full((pad,), N, jnp.int32)])
  src = src.reshape(NW, ep // (ch * nbuf), nbuf, ch)
  dst = dst.reshape(NW, ep // (ch * nbuf), nbuf, ch)

  yp = _make_segsum(N, E, D)(src, dst, x)

  vmem = pl.BlockSpec(memory_space=pltpu.VMEM)
  smem = pl.BlockSpec(memory_space=pltpu.SMEM)
  out = pl.pallas_call(
      _dense_body,
      out_shape=jax.ShapeDtypeStruct((N, D), jnp.float32),
      in_specs=[vmem, vmem, vmem, vmem, vmem, vmem, smem, vmem, vmem],
      out_specs=vmem,
  )(yp, x, W1, b1, W2, b2, eps, gamma, beta)
  return out


# speedup vs baseline: 1.0187x; 1.0187x over previous
import functools

import jax
import jax.numpy as jnp
from jax import lax
from jax.experimental import pallas as pl
from jax.experimental.pallas import tpu as pltpu
from jax.experimental.pallas import tpu_sc as plsc

NC = 2
NS = 16
NW = NC * NS


def _make_segsum(N, E, D):
  ch = 40
  nbuf = 7
  ep = -(-(E // NW) // (ch * nbuf)) * ch * nbuf
  npass = ep // (ch * nbuf)
  na = -(-N // ch) * ch + ch
  rch = 40
  nrc = N // rch
  nrc_per_tile = -(-nrc // NS)
  nzc = na // ch
  nzc_per_tile = -(-nzc // NS)

  mesh = plsc.VectorSubcoreMesh(core_axis_name="c", subcore_axis_name="s")

  @functools.partial(
      pl.kernel,
      out_type=jax.ShapeDtypeStruct((NC, N, D), jnp.float32),
      mesh=mesh,
      scratch_types=[
          [pltpu.VMEM((nbuf, ch), jnp.int32)] * 2,
          [pltpu.VMEM((nbuf, ch), jnp.int32)] * 2,
          [pltpu.VMEM((ch, D), jnp.float32)] * nbuf,
          pltpu.VMEM_SHARED((na, D), jnp.float32),
          [pltpu.SemaphoreType.DMA] * 2,
          [pltpu.SemaphoreType.DMA] * nbuf,
          [pltpu.SemaphoreType.DMA] * nbuf,
      ],
  )
  def segsum(src_hbm, dst_hbm, x_hbm, out_hbm, sidxb, didxb, rows, yacc,
             isem, gsem, ssem):
    c = lax.axis_index("c")
    s = lax.axis_index("s")
    wid = c * NS + s

    pltpu.async_copy(src_hbm.at[wid, 0], sidxb[0], isem[0])
    pltpu.async_copy(dst_hbm.at[wid, 0], didxb[0], isem[0])

    zv = jnp.zeros((16,), jnp.float32)

    def zrow(r, carry):
      for k in range(D // 16):
        rows[0][r, pl.ds(k * 16, 16)] = zv
      return carry

    lax.fori_loop(0, ch, zrow, 0)

    for z in range(nzc_per_tile):
      ci = s + NS * z

      @pl.when(ci < nzc)
      def _():
        pltpu.sync_copy(rows[0],
                        yacc.at[pl.ds(pl.multiple_of(ci * ch, 8), ch)])

    plsc.subcore_barrier()

    def two_passes(u, carry):
      for q in range(2):
        t = 2 * u + q
        nxt = 1 - q

        @pl.when(t + 1 < npass)
        def _():
          pltpu.async_copy(src_hbm.at[wid, t + 1], sidxb[nxt], isem[nxt])
          pltpu.async_copy(dst_hbm.at[wid, t + 1], didxb[nxt], isem[nxt])

        pltpu.make_async_copy(src_hbm.at[wid, t], sidxb[q], isem[q]).wait()
        pltpu.make_async_copy(dst_hbm.at[wid, t], didxb[q], isem[q]).wait()

        gh = [pltpu.async_copy(x_hbm.at[sidxb[q].at[b]], rows[b], gsem[b])
              for b in range(nbuf)]
        sh = []
        for b in range(nbuf):
          gh[b].wait()
          sh.append(pltpu.async_copy(rows[b], yacc.at[didxb[q].at[b]],
                                     ssem[b], add=True))
        for b in range(nbuf):
          sh[b].wait()
      return carry

    lax.fori_loop(0, npass // 2, two_passes, 0)

    plsc.subcore_barrier()

    for z in range(nrc_per_tile):
      ci = s + NS * z

      @pl.when(ci < nrc)
      def _():
        r0 = pl.multiple_of(ci * rch, 8)
        pltpu.sync_copy(yacc.at[pl.ds(r0, rch)], out_hbm.at[c, pl.ds(r0, rch)])

  return segsum


def _dense_body(yp_ref, x_ref, w1_ref, b1_ref, w2_ref, b2_ref, eps_ref,
                gamma_ref, beta_ref, o_ref):
  n = x_ref.shape[0]
  y = yp_ref[0] + yp_ref[1]
  h = y + (1.0 + eps_ref[0]) * x_ref[...]
  h = lax.dot_general(h, w1_ref[...], (((1,), (1,)), ((), ())),
                      preferred_element_type=jnp.float32)
  h = jnp.maximum(h + b1_ref[...][None, :], 0.0)
  h = lax.dot_general(h, w2_ref[...], (((1,), (1,)), ((), ())),
                      preferred_element_type=jnp.float32)
  h = h + b2_ref[...][None, :]
  mean = jnp.sum(h, axis=0, keepdims=True) * (1.0 / n)
  d = h - mean
  var = jnp.sum(d * d, axis=0, keepdims=True) * (1.0 / n)
  o_ref[...] = d * lax.rsqrt(var + 1e-5) * gamma_ref[...][None, :] \
      + beta_ref[...][None, :]


def kernel(x, edge_index, W1, b1, W2, b2, eps, gamma, beta):
  N, D = x.shape
  E = edge_index.shape[1]
  ch, nbuf = 40, 7
  ep = -(-(E // NW) // (ch * nbuf)) * ch * nbuf
  pad = ep * NW - E
  src = jnp.concatenate([edge_index[0], jnp.zeros((pad,), jnp.int32)])
  dst = jnp.concatenate([edge_index[1], jnp.full((pad,), N, jnp.int32)])
  src = src.reshape(NW, ep // (ch * nbuf), nbuf, ch)
  dst = dst.reshape(NW, ep // (ch * nbuf), nbuf, ch)

  yp = _make_segsum(N, E, D)(src, dst, x)

  vmem = pl.BlockSpec(memory_space=pltpu.VMEM)
  smem = pl.BlockSpec(memory_space=pltpu.SMEM)
  out = pl.pallas_call(
      _dense_body,
      out_shape=jax.ShapeDtypeStruct((N, D), jnp.float32),
      in_specs=[vmem, vmem, vmem, vmem, vmem, vmem, smem, vmem, vmem],
      out_specs=vmem,
  )(yp, x, W1, b1, W2, b2, eps, gamma, beta)
  return out


# speedup vs baseline: 1.6253x; 1.5953x over previous
import functools

import jax
import jax.numpy as jnp
from jax import lax
from jax.experimental import pallas as pl
from jax.experimental.pallas import tpu as pltpu
from jax.experimental.pallas import tpu_sc as plsc

NC = 2
NS = 16
NW = NC * NS


def _make_segsum(N, E, D):
  ch = 40
  nbuf = 7
  ep = -(-(E // NW) // (ch * nbuf)) * ch * nbuf
  npass = ep // (ch * nbuf)
  na = -(-N // ch) * ch + ch
  rch = 40
  nrc = N // rch
  nrc_per_tile = -(-nrc // NS)
  nzc = na // ch
  nzc_per_tile = -(-nzc // NS)

  mesh = plsc.VectorSubcoreMesh(core_axis_name="c", subcore_axis_name="s")

  @functools.partial(
      pl.kernel,
      out_type=jax.ShapeDtypeStruct((NC, N, D), jnp.float32),
      mesh=mesh,
      scratch_types=[
          [pltpu.VMEM((nbuf, ch), jnp.int32)] * 2,
          [pltpu.VMEM((nbuf, ch), jnp.int32)] * 2,
          [pltpu.VMEM((ch, D), jnp.float32)] * nbuf,
          pltpu.VMEM_SHARED((na, D), jnp.float32),
          [pltpu.SemaphoreType.DMA] * 2,
          [pltpu.SemaphoreType.DMA] * nbuf,
          [pltpu.SemaphoreType.DMA] * nbuf,
      ],
  )
  def segsum(src_hbm, dst_hbm, x_hbm, out_hbm, sidxb, didxb, rows, yacc,
             isem, gsem, ssem):
    c = lax.axis_index("c")
    s = lax.axis_index("s")
    wid = c * NS + s

    pltpu.async_copy(src_hbm.at[wid, 0], sidxb[0], isem[0])
    pltpu.async_copy(dst_hbm.at[wid, 0], didxb[0], isem[0])

    zv = jnp.zeros((16,), jnp.float32)

    def zrow(r, carry):
      for k in range(D // 16):
        rows[0][r, pl.ds(k * 16, 16)] = zv
      return carry

    lax.fori_loop(0, ch, zrow, 0)

    for z in range(nzc_per_tile):
      ci = s + NS * z

      @pl.when(ci < nzc)
      def _():
        pltpu.sync_copy(rows[0],
                        yacc.at[pl.ds(pl.multiple_of(ci * ch, 8), ch)])

    plsc.subcore_barrier()

    def two_passes(u, carry):
      for q in range(2):
        t = 2 * u + q
        nxt = 1 - q

        @pl.when(t + 1 < npass)
        def _():
          pltpu.async_copy(src_hbm.at[wid, t + 1], sidxb[nxt], isem[nxt])
          pltpu.async_copy(dst_hbm.at[wid, t + 1], didxb[nxt], isem[nxt])

        pltpu.make_async_copy(src_hbm.at[wid, t], sidxb[q], isem[q]).wait()
        pltpu.make_async_copy(dst_hbm.at[wid, t], didxb[q], isem[q]).wait()

        gh = [pltpu.async_copy(x_hbm.at[sidxb[q].at[b]], rows[b], gsem[b])
              for b in range(nbuf)]
        sh = []
        for b in range(nbuf):
          gh[b].wait()
          sh.append(pltpu.async_copy(rows[b], yacc.at[didxb[q].at[b]],
                                     ssem[b], add=True))
        for b in range(nbuf):
          sh[b].wait()
      return carry

    lax.fori_loop(0, npass // 2, two_passes, 0)

    plsc.subcore_barrier()

    for z in range(nrc_per_tile):
      ci = s + NS * z

      @pl.when(ci < nrc)
      def _():
        r0 = pl.multiple_of(ci * rch, 8)
        pltpu.sync_copy(yacc.at[pl.ds(r0, rch)], out_hbm.at[c, pl.ds(r0, rch)])

  return segsum


def _dense_body(yp_ref, x_ref, w1_ref, b1_ref, w2_ref, b2_ref, eps_ref,
                gamma_ref, beta_ref, o_ref):
  n = x_ref.shape[0]
  y = yp_ref[0] + yp_ref[1]
  h = y + (1.0 + eps_ref[0]) * x_ref[...]
  h = lax.dot_general(h, w1_ref[...], (((1,), (1,)), ((), ())),
                      preferred_element_type=jnp.float32)
  h = jnp.maximum(h + b1_ref[...][None, :], 0.0)
  h = lax.dot_general(h, w2_ref[...], (((1,), (1,)), ((), ())),
                      preferred_element_type=jnp.float32)
  h = h + b2_ref[...][None, :]
  mean = jnp.sum(h, axis=0, keepdims=True) * (1.0 / n)
  d = h - mean
  var = jnp.sum(d * d, axis=0, keepdims=True) * (1.0 / n)
  o_ref[...] = d * lax.rsqrt(var + 1e-5) * gamma_ref[...][None, :] \
      + beta_ref[...][None, :]


def kernel(x, edge_index, W1, b1, W2, b2, eps, gamma, beta):
  N, D = x.shape
  E = edge_index.shape[1]
  ch, nbuf = 40, 7
  ep = -(-(E // NW) // (ch * nbuf)) * ch * nbuf
  pad = ep * NW - E
  na_pad = (-(-N // ch) * ch + ch) - N
  pidx = jnp.arange(pad, dtype=jnp.int32)
  src = jnp.concatenate([edge_index[0], pidx % N])
  dst = jnp.concatenate([edge_index[1], N + pidx % na_pad])
  src = src.reshape(NW, ep // (ch * nbuf), nbuf, ch)
  dst = dst.reshape(NW, ep // (ch * nbuf), nbuf, ch)

  yp = _make_segsum(N, E, D)(src, dst, x)

  vmem = pl.BlockSpec(memory_space=pltpu.VMEM)
  smem = pl.BlockSpec(memory_space=pltpu.SMEM)
  out = pl.pallas_call(
      _dense_body,
      out_shape=jax.ShapeDtypeStruct((N, D), jnp.float32),
      in_specs=[vmem, vmem, vmem, vmem, vmem, vmem, smem, vmem, vmem],
      out_specs=vmem,
  )(yp, x, W1, b1, W2, b2, eps, gamma, beta)
  return out


# speedup vs baseline: 1.6921x; 1.0411x over previous
import functools

import jax
import jax.numpy as jnp
from jax import lax
from jax.experimental import pallas as pl
from jax.experimental.pallas import tpu as pltpu
from jax.experimental.pallas import tpu_sc as plsc

NC = 2
NS = 16
NW = NC * NS


def _make_segsum(N, E, D):
  ch = 40
  nbuf = 8
  ep = -(-(E // NW) // (ch * nbuf)) * ch * nbuf
  npass = ep // (ch * nbuf)
  na = -(-N // ch) * ch + ch
  rch = 40
  nrc = N // rch
  nrc_per_tile = -(-nrc // NS)
  nzc = na // ch
  nzc_per_tile = -(-nzc // NS)

  mesh = plsc.VectorSubcoreMesh(core_axis_name="c", subcore_axis_name="s")

  @functools.partial(
      pl.kernel,
      out_type=jax.ShapeDtypeStruct((NC, N, D), jnp.float32),
      mesh=mesh,
      scratch_types=[
          [pltpu.VMEM((nbuf, ch), jnp.int32)] * 2,
          [pltpu.VMEM((nbuf, ch), jnp.int32)] * 2,
          [pltpu.VMEM((ch, D), jnp.float32)] * nbuf,
          pltpu.VMEM_SHARED((na, D), jnp.float32),
          [pltpu.SemaphoreType.DMA] * 2,
          [pltpu.SemaphoreType.DMA] * nbuf,
          [pltpu.SemaphoreType.DMA] * nbuf,
      ],
  )
  def segsum(src_hbm, dst_hbm, x_hbm, out_hbm, sidxb, didxb, rows, yacc,
             isem, gsem, ssem):
    c = lax.axis_index("c")
    s = lax.axis_index("s")
    wid = c * NS + s

    pltpu.async_copy(src_hbm.at[wid, 0], sidxb[0], isem[0])
    pltpu.async_copy(dst_hbm.at[wid, 0], didxb[0], isem[0])

    zv = jnp.zeros((16,), jnp.float32)

    def zrow(r, carry):
      for k in range(D // 16):
        rows[0][r, pl.ds(k * 16, 16)] = zv
      return carry

    lax.fori_loop(0, ch, zrow, 0)

    for z in range(nzc_per_tile):
      ci = s + NS * z

      @pl.when(ci < nzc)
      def _():
        pltpu.sync_copy(rows[0],
                        yacc.at[pl.ds(pl.multiple_of(ci * ch, 8), ch)])

    plsc.subcore_barrier()

    def two_passes(u, carry):
      for q in range(2):
        t = 2 * u + q
        nxt = 1 - q

        @pl.when(t + 1 < npass)
        def _():
          pltpu.async_copy(src_hbm.at[wid, t + 1], sidxb[nxt], isem[nxt])
          pltpu.async_copy(dst_hbm.at[wid, t + 1], didxb[nxt], isem[nxt])

        pltpu.make_async_copy(src_hbm.at[wid, t], sidxb[q], isem[q]).wait()
        pltpu.make_async_copy(dst_hbm.at[wid, t], didxb[q], isem[q]).wait()

        gh = [pltpu.async_copy(x_hbm.at[sidxb[q].at[b]], rows[b], gsem[b])
              for b in range(nbuf)]
        sh = []
        for b in range(nbuf):
          gh[b].wait()
          sh.append(pltpu.async_copy(rows[b], yacc.at[didxb[q].at[b]],
                                     ssem[b], add=True))
        for b in range(nbuf):
          sh[b].wait()
      return carry

    lax.fori_loop(0, npass // 2, two_passes, 0)

    plsc.subcore_barrier()

    for z in range(nrc_per_tile):
      ci = s + NS * z

      @pl.when(ci < nrc)
      def _():
        r0 = pl.multiple_of(ci * rch, 8)
        pltpu.sync_copy(yacc.at[pl.ds(r0, rch)], out_hbm.at[c, pl.ds(r0, rch)])

  return segsum


def _dense_body(yp_ref, x_ref, w1_ref, b1_ref, w2_ref, b2_ref, eps_ref,
                gamma_ref, beta_ref, o_ref):
  n = x_ref.shape[0]
  y = yp_ref[0] + yp_ref[1]
  h = y + (1.0 + eps_ref[0]) * x_ref[...]
  h = lax.dot_general(h, w1_ref[...], (((1,), (1,)), ((), ())),
                      preferred_element_type=jnp.float32)
  h = jnp.maximum(h + b1_ref[...][None, :], 0.0)
  h = lax.dot_general(h, w2_ref[...], (((1,), (1,)), ((), ())),
                      preferred_element_type=jnp.float32)
  h = h + b2_ref[...][None, :]
  mean = jnp.sum(h, axis=0, keepdims=True) * (1.0 / n)
  d = h - mean
  var = jnp.sum(d * d, axis=0, keepdims=True) * (1.0 / n)
  o_ref[...] = d * lax.rsqrt(var + 1e-5) * gamma_ref[...][None, :] \
      + beta_ref[...][None, :]


def kernel(x, edge_index, W1, b1, W2, b2, eps, gamma, beta):
  N, D = x.shape
  E = edge_index.shape[1]
  ch, nbuf = 40, 8
  ep = -(-(E // NW) // (ch * nbuf)) * ch * nbuf
  pad = ep * NW - E
  na_pad = (-(-N // ch) * ch + ch) - N
  pidx = jnp.arange(pad, dtype=jnp.int32)
  src = jnp.concatenate([edge_index[0], pidx % N])
  dst = jnp.concatenate([edge_index[1], N + pidx % na_pad])
  src = src.reshape(NW, ep // (ch * nbuf), nbuf, ch)
  dst = dst.reshape(NW, ep // (ch * nbuf), nbuf, ch)

  yp = _make_segsum(N, E, D)(src, dst, x)

  vmem = pl.BlockSpec(memory_space=pltpu.VMEM)
  smem = pl.BlockSpec(memory_space=pltpu.SMEM)
  out = pl.pallas_call(
      _dense_body,
      out_shape=jax.ShapeDtypeStruct((N, D), jnp.float32),
      in_specs=[vmem, vmem, vmem, vmem, vmem, vmem, smem, vmem, vmem],
      out_specs=vmem,
  )(yp, x, W1, b1, W2, b2, eps, gamma, beta)
  return out


# speedup vs baseline: 1.6953x; 1.0019x over previous
import functools

import jax
import jax.numpy as jnp
from jax import lax
from jax.experimental import pallas as pl
from jax.experimental.pallas import tpu as pltpu
from jax.experimental.pallas import tpu_sc as plsc

NC = 2
NS = 16
NW = NC * NS


def _make_segsum(N, E, D):
  ch = 40
  nbuf = 4
  nset = 2
  ep = -(-(E // NW) // (ch * nbuf * nset)) * ch * nbuf * nset
  npass = ep // (ch * nbuf)
  na = -(-N // ch) * ch + 6 * ch
  rch = 40
  nrc = N // rch
  nrc_per_tile = -(-nrc // NS)
  nzc = na // ch
  nzc_per_tile = -(-nzc // NS)

  mesh = plsc.VectorSubcoreMesh(core_axis_name="c", subcore_axis_name="s")

  @functools.partial(
      pl.kernel,
      out_type=jax.ShapeDtypeStruct((NC, N, D), jnp.float32),
      mesh=mesh,
      scratch_types=[
          [pltpu.VMEM((nbuf, ch), jnp.int32)] * 2,
          [pltpu.VMEM((nbuf, ch), jnp.int32)] * 4,
          [pltpu.VMEM((ch, D), jnp.float32)] * (nbuf * nset),
          pltpu.VMEM_SHARED((na, D), jnp.float32),
          [pltpu.SemaphoreType.DMA] * 2,
          [pltpu.SemaphoreType.DMA] * 4,
          [pltpu.SemaphoreType.DMA] * (nbuf * nset),
          [pltpu.SemaphoreType.DMA] * (nbuf * nset),
      ],
  )
  def segsum(src_hbm, dst_hbm, x_hbm, out_hbm, sidxb, didxb, rows, yacc,
             isem, dsem, gsem, ssem):
    c = lax.axis_index("c")
    s = lax.axis_index("s")
    wid = c * NS + s

    pltpu.async_copy(src_hbm.at[wid, 0], sidxb[0], isem[0])
    pltpu.async_copy(dst_hbm.at[wid, 0], didxb[0], dsem[0])

    zv = jnp.zeros((16,), jnp.float32)

    def zrow(r, carry):
      for k in range(D // 16):
        rows[0][r, pl.ds(k * 16, 16)] = zv
      return carry

    lax.fori_loop(0, ch, zrow, 0)

    for z in range(nzc_per_tile):
      ci = s + NS * z

      @pl.when(ci < nzc)
      def _():
        pltpu.sync_copy(rows[0],
                        yacc.at[pl.ds(pl.multiple_of(ci * ch, 8), ch)])

    plsc.subcore_barrier()

    def four_passes(v, carry):
      for q in range(4):
        t = 4 * v + q
        p = q % 2
        R = rows[p * nbuf:(p + 1) * nbuf]
        G = gsem[p * nbuf:(p + 1) * nbuf]
        S = ssem[p * nbuf:(p + 1) * nbuf]
        si, di = q % 2, q % 4
        si_n, di_n = (q + 1) % 2, (q + 1) % 4

        @pl.when(t + 1 < npass)
        def _():
          pltpu.async_copy(src_hbm.at[wid, t + 1], sidxb[si_n], isem[si_n])
          pltpu.async_copy(dst_hbm.at[wid, t + 1], didxb[di_n], dsem[di_n])

        pltpu.make_async_copy(src_hbm.at[wid, t], sidxb[si], isem[si]).wait()
        pltpu.make_async_copy(dst_hbm.at[wid, t], didxb[di], dsem[di]).wait()

        @pl.when(t >= 2)
        def _():
          for b in range(nbuf):
            pltpu.make_async_copy(x_hbm.at[sidxb[si].at[b]], R[b],
                                  S[b]).wait()

        gh = [pltpu.async_copy(x_hbm.at[sidxb[si].at[b]], R[b], G[b])
              for b in range(nbuf)]
        for b in range(nbuf):
          gh[b].wait()
          pltpu.async_copy(R[b], yacc.at[didxb[di].at[b]], S[b], add=True)
      return carry

    lax.fori_loop(0, npass // 4, four_passes, 0)

    for b in range(nbuf * nset):
      pltpu.make_async_copy(x_hbm.at[sidxb[0].at[0]], rows[b],
                            ssem[b]).wait()

    plsc.subcore_barrier()

    for z in range(nrc_per_tile):
      ci = s + NS * z

      @pl.when(ci < nrc)
      def _():
        r0 = pl.multiple_of(ci * rch, 8)
        pltpu.sync_copy(yacc.at[pl.ds(r0, rch)], out_hbm.at[c, pl.ds(r0, rch)])

  return segsum


def _dense_body(yp_ref, x_ref, w1_ref, b1_ref, w2_ref, b2_ref, eps_ref,
                gamma_ref, beta_ref, o_ref):
  n = x_ref.shape[0]
  y = yp_ref[0] + yp_ref[1]
  h = y + (1.0 + eps_ref[0]) * x_ref[...]
  h = lax.dot_general(h, w1_ref[...], (((1,), (1,)), ((), ())),
                      preferred_element_type=jnp.float32)
  h = jnp.maximum(h + b1_ref[...][None, :], 0.0)
  h = lax.dot_general(h, w2_ref[...], (((1,), (1,)), ((), ())),
                      preferred_element_type=jnp.float32)
  h = h + b2_ref[...][None, :]
  mean = jnp.sum(h, axis=0, keepdims=True) * (1.0 / n)
  d = h - mean
  var = jnp.sum(d * d, axis=0, keepdims=True) * (1.0 / n)
  o_ref[...] = d * lax.rsqrt(var + 1e-5) * gamma_ref[...][None, :] \
      + beta_ref[...][None, :]


def kernel(x, edge_index, W1, b1, W2, b2, eps, gamma, beta):
  N, D = x.shape
  E = edge_index.shape[1]
  ch, nbuf, nset = 40, 4, 2
  ep = -(-(E // NW) // (ch * nbuf * nset)) * ch * nbuf * nset
  pad = ep * NW - E
  na_pad = (-(-N // ch) * ch + 6 * ch) - N
  pidx = jnp.arange(pad, dtype=jnp.int32)
  src = jnp.concatenate([edge_index[0], pidx % N])
  dst = jnp.concatenate([edge_index[1], N + pidx % na_pad])
  src = src.reshape(NW, ep // (ch * nbuf), nbuf, ch)
  dst = dst.reshape(NW, ep // (ch * nbuf), nbuf, ch)

  yp = _make_segsum(N, E, D)(src, dst, x)

  vmem = pl.BlockSpec(memory_space=pltpu.VMEM)
  smem = pl.BlockSpec(memory_space=pltpu.SMEM)
  out = pl.pallas_call(
      _dense_body,
      out_shape=jax.ShapeDtypeStruct((N, D), jnp.float32),
      in_specs=[vmem, vmem, vmem, vmem, vmem, vmem, smem, vmem, vmem],
      out_specs=vmem,
  )(yp, x, W1, b1, W2, b2, eps, gamma, beta)
  return out


# speedup vs baseline: 1.8629x; 1.0988x over previous
import functools

import jax
import jax.numpy as jnp
from jax import lax
from jax.experimental import pallas as pl
from jax.experimental.pallas import tpu as pltpu
from jax.experimental.pallas import tpu_sc as plsc

NC = 2
NS = 16
NW = NC * NS


def _make_segsum(N, E, D):
  ch = 40
  nbuf = 4
  nset = 2
  ep = E // NW
  pe = ch * nbuf
  nfull = ep // pe
  nmain = nfull // 4 * 4
  if nmain == nfull and nmain > 0:
    nmain -= 4
  tail = ep - nfull * pe
  ntail = tail // ch
  assert ep % ch == 0
  rch = 40
  nrc = N // rch
  nrc_per_tile = -(-nrc // NS)

  mesh = plsc.VectorSubcoreMesh(core_axis_name="c", subcore_axis_name="s")

  @functools.partial(
      pl.kernel,
      out_type=jax.ShapeDtypeStruct((NC, N, D), jnp.float32),
      mesh=mesh,
      scratch_types=[
          [pltpu.VMEM((pe,), jnp.int32)] * 2,
          [[pltpu.VMEM((ch,), jnp.int32)] * nbuf] * 4,
          [pltpu.VMEM((ch, D), jnp.float32)] * (nbuf * nset),
          pltpu.VMEM_SHARED((N, D), jnp.float32),
          [pltpu.SemaphoreType.DMA] * 2,
          [pltpu.SemaphoreType.DMA] * 4,
          [pltpu.SemaphoreType.DMA] * (nbuf * nset),
          [pltpu.SemaphoreType.DMA] * (nbuf * nset),
      ],
  )
  def segsum(src_hbm, dst_hbm, x_hbm, out_hbm, sidxb, didxb, rows, yacc,
             isem, dsem, gsem, ssem):
    c = lax.axis_index("c")
    s = lax.axis_index("s")
    wid = c * NS + s
    ebase = wid * ep

    def idx_prefetch(t, q, nch):
      si, di = q % 2, q % 4
      off = pl.multiple_of(ebase + t * pe, 8)
      if nch == nbuf:
        pltpu.async_copy(src_hbm.at[pl.ds(off, pe)], sidxb[si], isem[si])
      else:
        pltpu.async_copy(src_hbm.at[pl.ds(off, nch * ch)],
                         sidxb[si].at[pl.ds(0, nch * ch)], isem[si])
      for b in range(nch):
        boff = pl.multiple_of(ebase + t * pe + b * ch, 8)
        pltpu.async_copy(dst_hbm.at[pl.ds(boff, ch)], didxb[di][b], dsem[di])

    def idx_wait(q, nch):
      si, di = q % 2, q % 4
      if nch == nbuf:
        pltpu.make_async_copy(src_hbm.at[pl.ds(0, pe)], sidxb[si],
                              isem[si]).wait()
      else:
        pltpu.make_async_copy(src_hbm.at[pl.ds(0, nch * ch)],
                              sidxb[si].at[pl.ds(0, nch * ch)],
                              isem[si]).wait()
      for b in range(nch):
        pltpu.make_async_copy(dst_hbm.at[pl.ds(0, ch)], didxb[di][b],
                              dsem[di]).wait()

    def drain_scatters(q, nch):
      p = q % 2
      for b in range(nch):
        pltpu.make_async_copy(x_hbm.at[sidxb[0].at[pl.ds(0, ch)]],
                              rows[p * nbuf + b], ssem[p * nbuf + b]).wait()

    def do_pass(q, nch):
      p, si, di = q % 2, q % 2, q % 4
      R = rows[p * nbuf:p * nbuf + nch]
      gh = [pltpu.async_copy(
          x_hbm.at[sidxb[si].at[pl.ds(b * ch, ch)]], R[b],
          gsem[p * nbuf + b]) for b in range(nch)]
      for b in range(nch):
        gh[b].wait()
        pltpu.async_copy(R[b], yacc.at[didxb[di][b]],
                         ssem[p * nbuf + b], add=True)

    idx_prefetch(0, 0, nbuf)

    zv = jnp.zeros((16,), jnp.float32)

    def zrow(r, carry):
      for k in range(D // 16):
        rows[0][r, pl.ds(k * 16, 16)] = zv
      return carry

    lax.fori_loop(0, rch, zrow, 0)

    for z in range(nrc_per_tile):
      ci = s + NS * z

      @pl.when(ci < nrc)
      def _():
        pltpu.sync_copy(rows[0],
                        yacc.at[pl.ds(pl.multiple_of(ci * rch, 8), rch)])

    plsc.subcore_barrier()

    def four_passes(v, carry):
      for q in range(4):
        t = 4 * v + q
        idx_prefetch(t + 1, q + 1, nbuf)
        idx_wait(q, nbuf)

        @pl.when(t >= 2)
        def _():
          drain_scatters(q, nbuf)

        do_pass(q, nbuf)
      return carry

    lax.fori_loop(0, nmain // 4, four_passes, 0)

    for t in range(nmain, nfull):
      q = t % 4
      if t + 1 < nfull:
        idx_prefetch(t + 1, q + 1, nbuf)
      elif ntail:
        idx_prefetch(t + 1, q + 1, ntail)
      idx_wait(q, nbuf)
      drain_scatters(q, nbuf)
      do_pass(q, nbuf)

    if ntail:
      q = nfull % 4
      idx_wait(q, ntail)
      drain_scatters(q, ntail)
      do_pass(q, ntail)

    for b in range(nbuf * nset):
      pltpu.make_async_copy(x_hbm.at[sidxb[0].at[pl.ds(0, ch)]],
                            rows[b], ssem[b]).wait()

    plsc.subcore_barrier()

    for z in range(nrc_per_tile):
      ci = s + NS * z

      @pl.when(ci < nrc)
      def _():
        r0 = pl.multiple_of(ci * rch, 8)
        pltpu.sync_copy(yacc.at[pl.ds(r0, rch)], out_hbm.at[c, pl.ds(r0, rch)])

  return segsum


def _dense_body(yp_ref, x_ref, w1_ref, b1_ref, w2_ref, b2_ref, eps_ref,
                gamma_ref, beta_ref, o_ref):
  n = x_ref.shape[0]
  y = yp_ref[0] + yp_ref[1]
  h = y + (1.0 + eps_ref[0]) * x_ref[...]
  h = lax.dot_general(h, w1_ref[...], (((1,), (1,)), ((), ())),
                      preferred_element_type=jnp.float32)
  h = jnp.maximum(h + b1_ref[...][None, :], 0.0)
  h = lax.dot_general(h, w2_ref[...], (((1,), (1,)), ((), ())),
                      preferred_element_type=jnp.float32)
  h = h + b2_ref[...][None, :]
  mean = jnp.sum(h, axis=0, keepdims=True) * (1.0 / n)
  d = h - mean
  var = jnp.sum(d * d, axis=0, keepdims=True) * (1.0 / n)
  o_ref[...] = d * lax.rsqrt(var + 1e-5) * gamma_ref[...][None, :] \
      + beta_ref[...][None, :]


def kernel(x, edge_index, W1, b1, W2, b2, eps, gamma, beta):
  N, D = x.shape
  E = edge_index.shape[1]

  yp = _make_segsum(N, E, D)(edge_index[0], edge_index[1], x)

  vmem = pl.BlockSpec(memory_space=pltpu.VMEM)
  smem = pl.BlockSpec(memory_space=pltpu.SMEM)
  out = pl.pallas_call(
      _dense_body,
      out_shape=jax.ShapeDtypeStruct((N, D), jnp.float32),
      in_specs=[vmem, vmem, vmem, vmem, vmem, vmem, smem, vmem, vmem],
      out_specs=vmem,
  )(yp, x, W1, b1, W2, b2, eps, gamma, beta)
  return out
